# baseline (speedup 1.0000x reference)
import functools

import jax
import jax.numpy as jnp
from jax import lax
from jax.experimental import pallas as pl
from jax.experimental.pallas import tpu as pltpu
from jax.experimental.pallas import tpu_sc as plsc

LANES = 16


def kernel(target, context, target_table, context_table):
    if target.ndim == 2:
        target = jnp.squeeze(target, axis=1)
    target = target.astype(jnp.int32)
    context = context.astype(jnp.int32)

    B = target.shape[0]
    C = context.shape[1]
    E = target_table.shape[1]
    EV = E // LANES

    info = plsc.get_sparse_core_info()
    NW = info.num_cores * info.num_subcores
    PB = 64
    nb = B // NW
    NP = nb // PB

    ctx_flat = context.reshape(B * C)

    mesh = plsc.VectorSubcoreMesh(core_axis_name="c", subcore_axis_name="s")

    @functools.partial(
        pl.kernel,
        out_type=jax.ShapeDtypeStruct((B * C,), jnp.float32),
        mesh=mesh,
        compiler_params=pltpu.CompilerParams(needs_layout_passes=False),
        scratch_types=[
            pltpu.VMEM((nb,), jnp.int32),
            pltpu.VMEM((nb * C,), jnp.int32),
            pltpu.VMEM((2, PB, E), jnp.float32),
            pltpu.VMEM((2, PB * C, E), jnp.float32),
            pltpu.VMEM((PB * C,), jnp.float32),
            pltpu.VMEM((C * LANES, LANES), jnp.float32),
            pltpu.SemaphoreType.DMA,
            pltpu.SemaphoreType.DMA,
        ],
    )
    def sc_kernel(tgt_idx_hbm, ctx_idx_hbm, tgt_tab, ctx_tab, out_hbm,
                  tgt_idx, ctx_idx, w_rows, c_rows, out_v, acc_buf,
                  sem0, sem1):
        sems = (sem0, sem1)
        cid = lax.axis_index("c")
        sid = lax.axis_index("s")
        wid = sid * info.num_cores + cid

        pltpu.sync_copy(tgt_idx_hbm.at[pl.ds(wid * nb, nb)], tgt_idx)
        pltpu.sync_copy(ctx_idx_hbm.at[pl.ds(wid * nb * C, nb * C)], ctx_idx)

        def start(p, slot):
            pltpu.async_copy(
                tgt_tab.at[tgt_idx.at[pl.ds(p * PB, PB)]],
                w_rows.at[slot], sems[slot])
            for r in range(C):
                pltpu.async_copy(
                    ctx_tab.at[ctx_idx.at[pl.ds(p * PB * C + r * PB, PB)]],
                    c_rows.at[slot, pl.ds(r * PB, PB)], sems[slot])

        def drain(slot):
            pltpu.make_async_copy(
                tgt_tab.at[pl.ds(0, PB)], w_rows.at[slot], sems[slot]).wait()
            pltpu.make_async_copy(
                ctx_tab.at[pl.ds(0, PB * C)], c_rows.at[slot], sems[slot]).wait()

        lane = lax.iota(jnp.int32, LANES)

        def splat(v):
            return jnp.full((LANES,), v, jnp.int32)

        def compute(p, slot):
            def body(g, acc_carry):
                for i in range(LANES):
                    b = g * LANES + i
                    w = [w_rows[slot, b, pl.ds(LANES * j, LANES)]
                         for j in range(EV)]
                    for c in range(C):
                        row = b * C + c
                        lo = w[0] * c_rows[slot, row, pl.ds(0, LANES)]
                        hi = w[1] * c_rows[slot, row, pl.ds(LANES, LANES)]
                        for j in range(2, EV, 2):
                            lo = lo + w[j] * c_rows[slot, row, pl.ds(LANES * j, LANES)]
                            hi = hi + w[j + 1] * c_rows[slot, row, pl.ds(LANES * (j + 1), LANES)]
                        acc_buf[c * LANES + i, :] = lo + hi
                for c in range(C):
                    rows_idx = splat(c * LANES) + lane
                    res = plsc.load_gather(acc_buf, [rows_idx, splat(0)])
                    for j in range(1, LANES):
                        res = res + plsc.load_gather(acc_buf, [rows_idx, splat(j)])
                    idx = g * (LANES * C) + lane * C + c
                    plsc.store_scatter(out_v, [idx], res)
                return acc_carry

            lax.fori_loop(0, PB // LANES, body, 0)
            pltpu.sync_copy(
                out_v, out_hbm.at[pl.ds((wid * NP + p) * PB * C, PB * C)])

        start(0, 0)

        def outer(g, carry):
            for par in range(2):
                p = 2 * g + par

                @pl.when(p + 1 < NP)
                def _():
                    start(p + 1, 1 - par)

                drain(par)
                compute(p, par)
            return carry

        lax.fori_loop(0, NP // 2, outer, 0)

    out = sc_kernel(target, ctx_flat, target_table, context_table)
    return out.reshape(B, C)


# speedup vs baseline: 1.9254x; 1.9254x over previous
import functools

import jax
import jax.numpy as jnp
from jax import lax
from jax.experimental import pallas as pl
from jax.experimental.pallas import tpu as pltpu
from jax.experimental.pallas import tpu_sc as plsc

LANES = 16


def kernel(target, context, target_table, context_table):
    if target.ndim == 2:
        target = jnp.squeeze(target, axis=1)
    target = target.astype(jnp.int32)
    context = context.astype(jnp.int32)

    B = target.shape[0]
    C = context.shape[1]
    E = target_table.shape[1]
    EV = E // LANES

    info = plsc.get_sparse_core_info()
    NW = info.num_cores * info.num_subcores
    PB = 32
    nb = B // NW
    NP = nb // PB
    NG = PB // LANES

    ctx_flat = context.reshape(B * C)

    mesh = plsc.VectorSubcoreMesh(core_axis_name="c", subcore_axis_name="s")

    @functools.partial(
        pl.kernel,
        out_type=jax.ShapeDtypeStruct((B * C,), jnp.float32),
        mesh=mesh,
        compiler_params=pltpu.CompilerParams(needs_layout_passes=False),
        scratch_types=[
            pltpu.VMEM((nb,), jnp.int32),
            pltpu.VMEM((nb * C,), jnp.int32),
            pltpu.VMEM((PB, E), jnp.float32),
            pltpu.VMEM((PB, E), jnp.float32),
            pltpu.VMEM((PB * C, E), jnp.float32),
            pltpu.VMEM((PB * C, E), jnp.float32),
            pltpu.VMEM((PB * C,), jnp.float32),
            pltpu.VMEM((C * PB, LANES), jnp.float32),
            pltpu.SemaphoreType.DMA,
            pltpu.SemaphoreType.DMA,
        ],
    )
    def sc_kernel(tgt_idx_hbm, ctx_idx_hbm, tgt_tab, ctx_tab, out_hbm,
                  tgt_idx, ctx_idx, w_rows0, w_rows1, c_rows0, c_rows1,
                  out_v, acc_all, sem0, sem1):
        sems = (sem0, sem1)
        w_bufs = (w_rows0, w_rows1)
        c_bufs = (c_rows0, c_rows1)
        cid = lax.axis_index("c")
        sid = lax.axis_index("s")
        wid = sid * info.num_cores + cid

        pltpu.sync_copy(tgt_idx_hbm.at[pl.ds(wid * nb, nb)], tgt_idx)
        pltpu.sync_copy(ctx_idx_hbm.at[pl.ds(wid * nb * C, nb * C)], ctx_idx)

        def start(p, slot):
            pltpu.async_copy(
                tgt_tab.at[tgt_idx.at[pl.ds(p * PB, PB)]],
                w_bufs[slot], sems[slot])
            for r in range(C):
                pltpu.async_copy(
                    ctx_tab.at[ctx_idx.at[pl.ds(p * PB * C + r * PB, PB)]],
                    c_bufs[slot].at[pl.ds(r * PB, PB)], sems[slot])

        def drain(slot):
            pltpu.make_async_copy(
                tgt_tab.at[pl.ds(0, PB)], w_bufs[slot], sems[slot]).wait()
            pltpu.make_async_copy(
                ctx_tab.at[pl.ds(0, PB * C)], c_bufs[slot], sems[slot]).wait()

        lane = lax.iota(jnp.int32, LANES)

        def splat(v):
            return jnp.full((LANES,), v, jnp.int32)

        def compute(p, slot):
            @plsc.parallel_loop(0, PB, unroll=2)
            def dot_body(b):
                w = [w_bufs[slot][b, pl.ds(LANES * j, LANES)]
                     for j in range(EV)]
                for c in range(C):
                    row = b * C + c
                    lo = w[0] * c_bufs[slot][row, pl.ds(0, LANES)]
                    hi = w[1] * c_bufs[slot][row, pl.ds(LANES, LANES)]
                    for j in range(2, EV, 2):
                        lo = lo + w[j] * c_bufs[slot][row, pl.ds(LANES * j, LANES)]
                        hi = hi + w[j + 1] * c_bufs[slot][row, pl.ds(LANES * (j + 1), LANES)]
                    acc_all[c * PB + b, :] = lo + hi

            @plsc.parallel_loop(0, NG * C, unroll=2)
            def red_body(t):
                g = t // C
                c = t % C
                rows_idx = splat(0) + c * PB + g * LANES + lane
                res = plsc.load_gather(acc_all, [rows_idx, splat(0)])
                for j in range(1, LANES):
                    res = res + plsc.load_gather(acc_all, [rows_idx, splat(j)])
                idx = g * (LANES * C) + lane * C + c
                plsc.store_scatter(out_v, [idx], res)

            pltpu.sync_copy(
                out_v, out_hbm.at[pl.ds((wid * NP + p) * PB * C, PB * C)])

        start(0, 0)

        def outer(g, carry):
            for par in range(2):
                p = 2 * g + par

                @pl.when(p + 1 < NP)
                def _():
                    start(p + 1, 1 - par)

                drain(par)
                compute(p, par)
            return carry

        lax.fori_loop(0, NP // 2, outer, 0)

    out = sc_kernel(target, ctx_flat, target_table, context_table)
    return out.reshape(B, C)


# speedup vs baseline: 1.9405x; 1.0078x over previous
import functools

import jax
import jax.numpy as jnp
from jax import lax
from jax.experimental import pallas as pl
from jax.experimental.pallas import tpu as pltpu
from jax.experimental.pallas import tpu_sc as plsc

LANES = 16


def kernel(target, context, target_table, context_table):
    if target.ndim == 2:
        target = jnp.squeeze(target, axis=1)
    target = target.astype(jnp.int32)
    context = context.astype(jnp.int32)

    B = target.shape[0]
    C = context.shape[1]
    E = target_table.shape[1]
    EV = E // LANES

    info = plsc.get_sparse_core_info()
    NW = info.num_cores * info.num_subcores
    PB = 32
    nb = B // NW
    NP = nb // PB
    NG = PB // LANES

    ctx_flat = context.reshape(B * C)

    mesh = plsc.VectorSubcoreMesh(core_axis_name="c", subcore_axis_name="s")

    @functools.partial(
        pl.kernel,
        out_type=jax.ShapeDtypeStruct((B * C,), jnp.float32),
        mesh=mesh,
        compiler_params=pltpu.CompilerParams(needs_layout_passes=False),
        scratch_types=[
            pltpu.VMEM((nb,), jnp.int32),
            pltpu.VMEM((nb * C,), jnp.int32),
            pltpu.VMEM((PB, E), jnp.float32),
            pltpu.VMEM((PB, E), jnp.float32),
            pltpu.VMEM((PB * C, E), jnp.float32),
            pltpu.VMEM((PB * C, E), jnp.float32),
            pltpu.VMEM((PB * C,), jnp.float32),
            pltpu.VMEM((PB * C,), jnp.float32),
            pltpu.VMEM((C * PB, LANES), jnp.float32),
            pltpu.SemaphoreType.DMA,
            pltpu.SemaphoreType.DMA,
            pltpu.SemaphoreType.DMA,
        ],
    )
    def sc_kernel(tgt_idx_hbm, ctx_idx_hbm, tgt_tab, ctx_tab, out_hbm,
                  tgt_idx, ctx_idx, w_rows0, w_rows1, c_rows0, c_rows1,
                  out_v0, out_v1, acc_all, sem0, sem1, osem):
        sems = (sem0, sem1)
        w_bufs = (w_rows0, w_rows1)
        c_bufs = (c_rows0, c_rows1)
        out_bufs = (out_v0, out_v1)
        cid = lax.axis_index("c")
        sid = lax.axis_index("s")
        wid = sid * info.num_cores + cid

        pltpu.sync_copy(tgt_idx_hbm.at[pl.ds(wid * nb, nb)], tgt_idx)
        pltpu.sync_copy(ctx_idx_hbm.at[pl.ds(wid * nb * C, nb * C)], ctx_idx)

        def start(p, slot):
            pltpu.async_copy(
                tgt_tab.at[tgt_idx.at[pl.ds(p * PB, PB)]],
                w_bufs[slot], sems[slot])
            for r in range(C):
                pltpu.async_copy(
                    ctx_tab.at[ctx_idx.at[pl.ds(p * PB * C + r * PB, PB)]],
                    c_bufs[slot].at[pl.ds(r * PB, PB)], sems[slot])

        def drain(slot):
            pltpu.make_async_copy(
                tgt_tab.at[pl.ds(0, PB)], w_bufs[slot], sems[slot]).wait()
            pltpu.make_async_copy(
                ctx_tab.at[pl.ds(0, PB * C)], c_bufs[slot], sems[slot]).wait()

        lane = lax.iota(jnp.int32, LANES)

        def splat(v):
            return jnp.full((LANES,), v, jnp.int32)

        def compute(p, slot):
            out_v = out_bufs[slot]

            @pl.when(p >= 2)
            def _():
                pltpu.make_async_copy(
                    out_v, out_hbm.at[pl.ds(0, PB * C)], osem).wait()

            @plsc.parallel_loop(0, PB, unroll=2)
            def dot_body(b):
                w = [w_bufs[slot][b, pl.ds(LANES * j, LANES)]
                     for j in range(EV)]
                for c in range(C):
                    row = b * C + c
                    lo = w[0] * c_bufs[slot][row, pl.ds(0, LANES)]
                    hi = w[1] * c_bufs[slot][row, pl.ds(LANES, LANES)]
                    for j in range(2, EV, 2):
                        lo = lo + w[j] * c_bufs[slot][row, pl.ds(LANES * j, LANES)]
                        hi = hi + w[j + 1] * c_bufs[slot][row, pl.ds(LANES * (j + 1), LANES)]
                    acc_all[c * PB + b, :] = lo + hi

            @plsc.parallel_loop(0, NG * C, unroll=2)
            def red_body(t):
                g = t // C
                c = t % C
                rows_idx = splat(0) + c * PB + g * LANES + lane
                res = plsc.load_gather(acc_all, [rows_idx, splat(0)])
                for j in range(1, LANES):
                    res = res + plsc.load_gather(acc_all, [rows_idx, splat(j)])
                idx = g * (LANES * C) + lane * C + c
                plsc.store_scatter(out_v, [idx], res)

            pltpu.async_copy(
                out_v, out_hbm.at[pl.ds((wid * NP + p) * PB * C, PB * C)],
                osem)

        start(0, 0)

        def outer(g, carry):
            for par in range(2):
                p = 2 * g + par

                @pl.when(p + 1 < NP)
                def _():
                    start(p + 1, 1 - par)

                drain(par)
                compute(p, par)
            return carry

        lax.fori_loop(0, NP // 2, outer, 0)

        for slot in range(2):
            pltpu.make_async_copy(
                out_bufs[slot], out_hbm.at[pl.ds(0, PB * C)], osem).wait()

    out = sc_kernel(target, ctx_flat, target_table, context_table)
    return out.reshape(B, C)


# speedup vs baseline: 2.0072x; 1.0344x over previous
import functools

import jax
import jax.numpy as jnp
from jax import lax
from jax.experimental import pallas as pl
from jax.experimental.pallas import tpu as pltpu
from jax.experimental.pallas import tpu_sc as plsc

LANES = 16


def kernel(target, context, target_table, context_table):
    if target.ndim == 2:
        target = jnp.squeeze(target, axis=1)
    target = target.astype(jnp.int32)
    context = context.astype(jnp.int32)

    B = target.shape[0]
    C = context.shape[1]
    E = target_table.shape[1]
    EV = E // LANES

    info = plsc.get_sparse_core_info()
    NW = info.num_cores * info.num_subcores
    PB = 32
    nb = B // NW
    NP = nb // PB
    NG = PB // LANES

    ctx_flat = context.reshape(B * C)

    mesh = plsc.VectorSubcoreMesh(core_axis_name="c", subcore_axis_name="s")

    @functools.partial(
        pl.kernel,
        out_type=jax.ShapeDtypeStruct((B * C,), jnp.float32),
        mesh=mesh,
        compiler_params=pltpu.CompilerParams(needs_layout_passes=False),
        scratch_types=[
            pltpu.VMEM((nb,), jnp.int32),
            pltpu.VMEM((nb * C,), jnp.int32),
            pltpu.VMEM((PB, E), jnp.float32),
            pltpu.VMEM((PB, E), jnp.float32),
            pltpu.VMEM((PB * C, E), jnp.float32),
            pltpu.VMEM((PB * C, E), jnp.float32),
            pltpu.VMEM((PB * C,), jnp.float32),
            pltpu.VMEM((PB * C,), jnp.float32),
            pltpu.VMEM((C * PB, LANES), jnp.float32),
            pltpu.SemaphoreType.DMA,
            pltpu.SemaphoreType.DMA,
            pltpu.SemaphoreType.DMA,
        ],
    )
    def sc_kernel(tgt_idx_hbm, ctx_idx_hbm, tgt_tab, ctx_tab, out_hbm,
                  tgt_idx, ctx_idx, w_rows0, w_rows1, c_rows0, c_rows1,
                  out_v0, out_v1, acc_all, sem0, sem1, osem):
        sems = (sem0, sem1)
        w_bufs = (w_rows0, w_rows1)
        c_bufs = (c_rows0, c_rows1)
        out_bufs = (out_v0, out_v1)
        cid = lax.axis_index("c")
        sid = lax.axis_index("s")
        wid = sid * info.num_cores + cid

        pltpu.sync_copy(tgt_idx_hbm.at[pl.ds(wid * nb, nb)], tgt_idx)
        pltpu.sync_copy(ctx_idx_hbm.at[pl.ds(wid * nb * C, nb * C)], ctx_idx)

        def start(p, slot):
            pltpu.async_copy(
                tgt_tab.at[tgt_idx.at[pl.ds(p * PB, PB)]],
                w_bufs[slot], sems[slot])
            for r in range(C):
                pltpu.async_copy(
                    ctx_tab.at[ctx_idx.at[pl.ds(p * PB * C + r * PB, PB)]],
                    c_bufs[slot].at[pl.ds(r * PB, PB)], sems[slot])

        def drain(slot):
            pltpu.make_async_copy(
                tgt_tab.at[pl.ds(0, PB)], w_bufs[slot], sems[slot]).wait()
            pltpu.make_async_copy(
                ctx_tab.at[pl.ds(0, PB * C)], c_bufs[slot], sems[slot]).wait()

        lane = lax.iota(jnp.int32, LANES)

        def splat(v):
            return jnp.full((LANES,), v, jnp.int32)

        def compute(p, slot):
            out_v = out_bufs[slot]

            @pl.when(p >= 2)
            def _():
                pltpu.make_async_copy(
                    out_v, out_hbm.at[pl.ds(0, PB * C)], osem).wait()

            @plsc.parallel_loop(0, PB, unroll=2)
            def dot_body(b):
                w = [w_bufs[slot][b, pl.ds(LANES * j, LANES)]
                     for j in range(EV)]
                for c in range(C):
                    row = b * C + c
                    lo = w[0] * c_bufs[slot][row, pl.ds(0, LANES)]
                    hi = w[1] * c_bufs[slot][row, pl.ds(LANES, LANES)]
                    for j in range(2, EV, 2):
                        lo = lo + w[j] * c_bufs[slot][row, pl.ds(LANES * j, LANES)]
                        hi = hi + w[j + 1] * c_bufs[slot][row, pl.ds(LANES * (j + 1), LANES)]
                    acc_all[c * PB + b, :] = lo + hi

            @plsc.parallel_loop(0, NG * C, unroll=2)
            def red_body(t):
                g = t // C
                c = t % C
                rows_idx = splat(0) + c * PB + g * LANES + lane
                cols = [plsc.load_gather(acc_all, [rows_idx, splat(j)])
                        for j in range(LANES)]
                while len(cols) > 1:
                    cols = [cols[k] + cols[k + 1]
                            for k in range(0, len(cols), 2)]
                res = cols[0]
                idx = g * (LANES * C) + lane * C + c
                plsc.store_scatter(out_v, [idx], res)

            pltpu.async_copy(
                out_v, out_hbm.at[pl.ds((wid * NP + p) * PB * C, PB * C)],
                osem)

        start(0, 0)

        def outer(g, carry):
            for par in range(2):
                p = 2 * g + par

                @pl.when(p + 1 < NP)
                def _():
                    start(p + 1, 1 - par)

                drain(par)
                compute(p, par)
            return carry

        lax.fori_loop(0, NP // 2, outer, 0)

        for slot in range(2):
            pltpu.make_async_copy(
                out_bufs[slot], out_hbm.at[pl.ds(0, PB * C)], osem).wait()

    out = sc_kernel(target, ctx_flat, target_table, context_table)
    return out.reshape(B, C)


# speedup vs baseline: 2.0099x; 1.0014x over previous
import functools

import jax
import jax.numpy as jnp
from jax import lax
from jax.experimental import pallas as pl
from jax.experimental.pallas import tpu as pltpu
from jax.experimental.pallas import tpu_sc as plsc

LANES = 16


def kernel(target, context, target_table, context_table):
    if target.ndim == 2:
        target = jnp.squeeze(target, axis=1)
    target = target.astype(jnp.int32)
    context = context.astype(jnp.int32)

    B = target.shape[0]
    C = context.shape[1]
    E = target_table.shape[1]
    EV = E // LANES

    info = plsc.get_sparse_core_info()
    NW = info.num_cores * info.num_subcores
    PB = 32
    nb = B // NW
    NP = nb // PB
    NG = PB // LANES

    ctx_flat = context.reshape(B * C)

    mesh = plsc.VectorSubcoreMesh(core_axis_name="c", subcore_axis_name="s")

    @functools.partial(
        pl.kernel,
        out_type=jax.ShapeDtypeStruct((B * C,), jnp.float32),
        mesh=mesh,
        compiler_params=pltpu.CompilerParams(needs_layout_passes=False),
        scratch_types=[
            pltpu.VMEM((nb,), jnp.int32),
            pltpu.VMEM((nb * C,), jnp.int32),
            pltpu.VMEM((PB, E), jnp.float32),
            pltpu.VMEM((PB, E), jnp.float32),
            pltpu.VMEM((PB * C, E), jnp.float32),
            pltpu.VMEM((PB * C, E), jnp.float32),
            pltpu.VMEM((PB * C,), jnp.float32),
            pltpu.VMEM((PB * C,), jnp.float32),
            pltpu.VMEM((C * PB, LANES + 1), jnp.float32),
            pltpu.SemaphoreType.DMA,
            pltpu.SemaphoreType.DMA,
            pltpu.SemaphoreType.DMA,
        ],
    )
    def sc_kernel(tgt_idx_hbm, ctx_idx_hbm, tgt_tab, ctx_tab, out_hbm,
                  tgt_idx, ctx_idx, w_rows0, w_rows1, c_rows0, c_rows1,
                  out_v0, out_v1, acc_all, sem0, sem1, osem):
        sems = (sem0, sem1)
        w_bufs = (w_rows0, w_rows1)
        c_bufs = (c_rows0, c_rows1)
        out_bufs = (out_v0, out_v1)
        cid = lax.axis_index("c")
        sid = lax.axis_index("s")
        wid = sid * info.num_cores + cid

        pltpu.sync_copy(tgt_idx_hbm.at[pl.ds(wid * nb, nb)], tgt_idx)
        pltpu.sync_copy(ctx_idx_hbm.at[pl.ds(wid * nb * C, nb * C)], ctx_idx)

        def start(p, slot):
            pltpu.async_copy(
                tgt_tab.at[tgt_idx.at[pl.ds(p * PB, PB)]],
                w_bufs[slot], sems[slot])
            for r in range(C):
                pltpu.async_copy(
                    ctx_tab.at[ctx_idx.at[pl.ds(p * PB * C + r * PB, PB)]],
                    c_bufs[slot].at[pl.ds(r * PB, PB)], sems[slot])

        def drain(slot):
            pltpu.make_async_copy(
                tgt_tab.at[pl.ds(0, PB)], w_bufs[slot], sems[slot]).wait()
            pltpu.make_async_copy(
                ctx_tab.at[pl.ds(0, PB * C)], c_bufs[slot], sems[slot]).wait()

        lane = lax.iota(jnp.int32, LANES)

        def splat(v):
            return jnp.full((LANES,), v, jnp.int32)

        def compute(p, slot):
            out_v = out_bufs[slot]

            @pl.when(p >= 2)
            def _():
                pltpu.make_async_copy(
                    out_v, out_hbm.at[pl.ds(0, PB * C)], osem).wait()

            @plsc.parallel_loop(0, PB, unroll=2)
            def dot_body(b):
                w = [w_bufs[slot][b, pl.ds(LANES * j, LANES)]
                     for j in range(EV)]
                for c in range(C):
                    row = b * C + c
                    lo = w[0] * c_bufs[slot][row, pl.ds(0, LANES)]
                    hi = w[1] * c_bufs[slot][row, pl.ds(LANES, LANES)]
                    for j in range(2, EV, 2):
                        lo = lo + w[j] * c_bufs[slot][row, pl.ds(LANES * j, LANES)]
                        hi = hi + w[j + 1] * c_bufs[slot][row, pl.ds(LANES * (j + 1), LANES)]
                    acc_all[c * PB + b, pl.ds(0, LANES)] = lo + hi

            @plsc.parallel_loop(0, NG * C, unroll=2)
            def red_body(t):
                g = t // C
                c = t % C
                rows_idx = splat(0) + c * PB + g * LANES + lane
                cols = [plsc.load_gather(acc_all, [rows_idx, splat(j)])
                        for j in range(LANES)]
                while len(cols) > 1:
                    cols = [cols[k] + cols[k + 1]
                            for k in range(0, len(cols), 2)]
                res = cols[0]
                idx = g * (LANES * C) + lane * C + c
                plsc.store_scatter(out_v, [idx], res)

            pltpu.async_copy(
                out_v, out_hbm.at[pl.ds((wid * NP + p) * PB * C, PB * C)],
                osem)

        start(0, 0)

        def outer(g, carry):
            for par in range(2):
                p = 2 * g + par

                @pl.when(p + 1 < NP)
                def _():
                    start(p + 1, 1 - par)

                drain(par)
                compute(p, par)
            return carry

        lax.fori_loop(0, NP // 2, outer, 0)

        for slot in range(2):
            pltpu.make_async_copy(
                out_bufs[slot], out_hbm.at[pl.ds(0, PB * C)], osem).wait()

    out = sc_kernel(target, ctx_flat, target_table, context_table)
    return out.reshape(B, C)


# speedup vs baseline: 2.1351x; 1.0623x over previous
import functools

import jax
import jax.numpy as jnp
from jax import lax
from jax.experimental import pallas as pl
from jax.experimental.pallas import tpu as pltpu
from jax.experimental.pallas import tpu_sc as plsc

LANES = 16


def kernel(target, context, target_table, context_table):
    if target.ndim == 2:
        target = jnp.squeeze(target, axis=1)
    target = target.astype(jnp.int32)
    context = context.astype(jnp.int32)

    B = target.shape[0]
    C = context.shape[1]
    E = target_table.shape[1]
    EV = E // LANES

    info = plsc.get_sparse_core_info()
    NW = info.num_cores * info.num_subcores
    PB = 32
    nb = B // NW
    NP = nb // PB
    NG = PB // LANES

    ctx_flat = context.reshape(B * C)

    mesh = plsc.VectorSubcoreMesh(core_axis_name="c", subcore_axis_name="s")

    @functools.partial(
        pl.kernel,
        out_type=jax.ShapeDtypeStruct((B * C,), jnp.float32),
        mesh=mesh,
        compiler_params=pltpu.CompilerParams(needs_layout_passes=False),
        scratch_types=[
            pltpu.VMEM((nb,), jnp.int32),
            pltpu.VMEM((nb * C,), jnp.int32),
            pltpu.VMEM((PB, E), jnp.float32),
            pltpu.VMEM((PB, E), jnp.float32),
            pltpu.VMEM((PB * C, E), jnp.float32),
            pltpu.VMEM((PB * C, E), jnp.float32),
            pltpu.VMEM((PB * C,), jnp.float32),
            pltpu.VMEM((PB * C,), jnp.float32),
            pltpu.VMEM((C * PB, LANES + 1), jnp.float32),
            pltpu.SemaphoreType.DMA,
            pltpu.SemaphoreType.DMA,
            pltpu.SemaphoreType.DMA,
        ],
    )
    def sc_kernel(tgt_idx_hbm, ctx_idx_hbm, tgt_tab, ctx_tab, out_hbm,
                  tgt_idx, ctx_idx, w_rows0, w_rows1, c_rows0, c_rows1,
                  out_v0, out_v1, acc_all, sem0, sem1, osem):
        sems = (sem0, sem1)
        w_bufs = (w_rows0, w_rows1)
        c_bufs = (c_rows0, c_rows1)
        out_bufs = (out_v0, out_v1)
        cid = lax.axis_index("c")
        sid = lax.axis_index("s")
        wid = sid * info.num_cores + cid

        pltpu.sync_copy(tgt_idx_hbm.at[pl.ds(wid * nb, nb)], tgt_idx)
        pltpu.sync_copy(ctx_idx_hbm.at[pl.ds(wid * nb * C, nb * C)], ctx_idx)

        def start(p, slot):
            pltpu.async_copy(
                tgt_tab.at[tgt_idx.at[pl.ds(p * PB, PB)]],
                w_bufs[slot], sems[slot])
            for r in range(C):
                pltpu.async_copy(
                    ctx_tab.at[ctx_idx.at[pl.ds(p * PB * C + r * PB, PB)]],
                    c_bufs[slot].at[pl.ds(r * PB, PB)], sems[slot])

        def drain(slot):
            pltpu.make_async_copy(
                tgt_tab.at[pl.ds(0, PB)], w_bufs[slot], sems[slot]).wait()
            pltpu.make_async_copy(
                ctx_tab.at[pl.ds(0, PB * C)], c_bufs[slot], sems[slot]).wait()

        lane = lax.iota(jnp.int32, LANES)

        def splat(v):
            return jnp.full((LANES,), v, jnp.int32)

        def compute(p, slot):
            out_v = out_bufs[slot]

            @pl.when(p >= 2)
            def _():
                pltpu.make_async_copy(
                    out_v, out_hbm.at[pl.ds(0, PB * C)], osem).wait()

            @plsc.parallel_loop(0, PB, unroll=2)
            def dot_body(b):
                w = [w_bufs[slot][b, pl.ds(LANES * j, LANES)]
                     for j in range(EV)]
                for c in range(C):
                    row = b * C + c
                    lo = w[0] * c_bufs[slot][row, pl.ds(0, LANES)]
                    hi = w[1] * c_bufs[slot][row, pl.ds(LANES, LANES)]
                    for j in range(2, EV, 2):
                        lo = lo + w[j] * c_bufs[slot][row, pl.ds(LANES * j, LANES)]
                        hi = hi + w[j + 1] * c_bufs[slot][row, pl.ds(LANES * (j + 1), LANES)]
                    acc = lo + hi
                    acc_all[c * PB + b, pl.ds(0, LANES)] = acc + lax.rev(acc, (0,))

            @plsc.parallel_loop(0, NG * C, unroll=2)
            def red_body(t):
                g = t // C
                c = t % C
                rows_idx = splat(0) + c * PB + g * LANES + lane
                cols = [plsc.load_gather(acc_all, [rows_idx, splat(j)])
                        for j in range(LANES // 2)]
                while len(cols) > 1:
                    cols = [cols[k] + cols[k + 1]
                            for k in range(0, len(cols), 2)]
                res = cols[0]
                idx = g * (LANES * C) + lane * C + c
                plsc.store_scatter(out_v, [idx], res)

            pltpu.async_copy(
                out_v, out_hbm.at[pl.ds((wid * NP + p) * PB * C, PB * C)],
                osem)

        start(0, 0)

        def outer(g, carry):
            for par in range(2):
                p = 2 * g + par

                @pl.when(p + 1 < NP)
                def _():
                    start(p + 1, 1 - par)

                drain(par)
                compute(p, par)
            return carry

        lax.fori_loop(0, NP // 2, outer, 0)

        for slot in range(2):
            pltpu.make_async_copy(
                out_bufs[slot], out_hbm.at[pl.ds(0, PB * C)], osem).wait()

    out = sc_kernel(target, ctx_flat, target_table, context_table)
    return out.reshape(B, C)


# speedup vs baseline: 2.4074x; 1.1275x over previous
import functools

import jax
import jax.numpy as jnp
from jax import lax
from jax.experimental import pallas as pl
from jax.experimental.pallas import tpu as pltpu
from jax.experimental.pallas import tpu_sc as plsc

LANES = 16


def kernel(target, context, target_table, context_table):
    if target.ndim == 2:
        target = jnp.squeeze(target, axis=1)
    target = target.astype(jnp.int32)
    context = context.astype(jnp.int32)

    B = target.shape[0]
    C = context.shape[1]
    E = target_table.shape[1]
    EV = E // LANES

    info = plsc.get_sparse_core_info()
    NW = info.num_cores * info.num_subcores
    PB = 32
    nb = B // NW
    NP = nb // PB
    NG = PB // LANES

    ctx_flat = context.reshape(B * C)

    mesh = plsc.VectorSubcoreMesh(core_axis_name="c", subcore_axis_name="s")

    @functools.partial(
        pl.kernel,
        out_type=jax.ShapeDtypeStruct((B, C), jnp.float32),
        mesh=mesh,
        compiler_params=pltpu.CompilerParams(needs_layout_passes=False),
        scratch_types=[
            pltpu.VMEM((nb,), jnp.int32),
            pltpu.VMEM((nb * C,), jnp.int32),
            pltpu.VMEM((PB, E), jnp.float32),
            pltpu.VMEM((PB, E), jnp.float32),
            pltpu.VMEM((PB * C, E), jnp.float32),
            pltpu.VMEM((PB * C, E), jnp.float32),
            pltpu.VMEM((PB, C), jnp.float32),
            pltpu.VMEM((PB, C), jnp.float32),
            pltpu.VMEM((C * PB, LANES + 1), jnp.float32),
            pltpu.SemaphoreType.DMA,
            pltpu.SemaphoreType.DMA,
            pltpu.SemaphoreType.DMA,
        ],
    )
    def sc_kernel(tgt_idx_hbm, ctx_idx_hbm, tgt_tab, ctx_tab, out_hbm,
                  tgt_idx, ctx_idx, w_rows0, w_rows1, c_rows0, c_rows1,
                  out_v0, out_v1, acc_all, sem0, sem1, osem):
        sems = (sem0, sem1)
        w_bufs = (w_rows0, w_rows1)
        c_bufs = (c_rows0, c_rows1)
        out_bufs = (out_v0, out_v1)
        cid = lax.axis_index("c")
        sid = lax.axis_index("s")
        wid = sid * info.num_cores + cid

        pltpu.sync_copy(tgt_idx_hbm.at[pl.ds(wid * nb, nb)], tgt_idx)
        pltpu.sync_copy(ctx_idx_hbm.at[pl.ds(wid * nb * C, nb * C)], ctx_idx)

        def start(p, slot):
            pltpu.async_copy(
                tgt_tab.at[tgt_idx.at[pl.ds(p * PB, PB)]],
                w_bufs[slot], sems[slot])
            for r in range(C):
                pltpu.async_copy(
                    ctx_tab.at[ctx_idx.at[pl.ds(p * PB * C + r * PB, PB)]],
                    c_bufs[slot].at[pl.ds(r * PB, PB)], sems[slot])

        def drain(slot):
            pltpu.make_async_copy(
                tgt_tab.at[pl.ds(0, PB)], w_bufs[slot], sems[slot]).wait()
            pltpu.make_async_copy(
                ctx_tab.at[pl.ds(0, PB * C)], c_bufs[slot], sems[slot]).wait()

        lane = lax.iota(jnp.int32, LANES)

        def splat(v):
            return jnp.full((LANES,), v, jnp.int32)

        def compute(p, slot):
            out_v = out_bufs[slot]

            @pl.when(p >= 2)
            def _():
                pltpu.make_async_copy(
                    out_v, out_hbm.at[pl.ds(0, PB)], osem).wait()

            @plsc.parallel_loop(0, PB, unroll=2)
            def dot_body(b):
                w = [w_bufs[slot][b, pl.ds(LANES * j, LANES)]
                     for j in range(EV)]
                for c in range(C):
                    row = b * C + c
                    lo = w[0] * c_bufs[slot][row, pl.ds(0, LANES)]
                    hi = w[1] * c_bufs[slot][row, pl.ds(LANES, LANES)]
                    for j in range(2, EV, 2):
                        lo = lo + w[j] * c_bufs[slot][row, pl.ds(LANES * j, LANES)]
                        hi = hi + w[j + 1] * c_bufs[slot][row, pl.ds(LANES * (j + 1), LANES)]
                    acc = lo + hi
                    acc_all[c * PB + b, pl.ds(0, LANES)] = acc + lax.rev(acc, (0,))

            @plsc.parallel_loop(0, NG * C, unroll=2)
            def red_body(t):
                g = t // C
                c = t % C
                rows_idx = splat(0) + c * PB + g * LANES + lane
                cols = [plsc.load_gather(acc_all, [rows_idx, splat(j)])
                        for j in range(LANES // 2)]
                while len(cols) > 1:
                    cols = [cols[k] + cols[k + 1]
                            for k in range(0, len(cols), 2)]
                res = cols[0]
                row_idx = splat(g * LANES) + lane
                plsc.store_scatter(out_v, [row_idx, splat(c)], res)

            pltpu.async_copy(
                out_v, out_hbm.at[pl.ds((wid * NP + p) * PB, PB)],
                osem)

        start(0, 0)

        def outer(g, carry):
            for par in range(2):
                p = 2 * g + par

                @pl.when(p + 1 < NP)
                def _():
                    start(p + 1, 1 - par)

                drain(par)
                compute(p, par)
            return carry

        lax.fori_loop(0, NP // 2, outer, 0)

        for slot in range(2):
            pltpu.make_async_copy(
                out_bufs[slot], out_hbm.at[pl.ds(0, PB)], osem).wait()

    return sc_kernel(target, ctx_flat, target_table, context_table)
